# baseline (speedup 1.0000x reference)
import functools

import jax
import jax.numpy as jnp
from jax import lax
from jax.experimental import pallas as pl
from jax.experimental.pallas import tpu as pltpu
from jax.experimental.pallas import tpu_sc as plsc

NUM_CORES = 2
NUM_SUBCORES = 16
NUM_WORKERS = NUM_CORES * NUM_SUBCORES

GATHER = 128
IT = 4
IPW = GATHER * IT
NJ = 200
KT, KR = 4, 8
D = KT * KR


def _sc_embedding_lookup(table, idx3):
    assert table.shape[1] == D and idx3.shape == (NJ, 128, 128)

    mesh = plsc.VectorSubcoreMesh(core_axis_name="c", subcore_axis_name="s")

    @functools.partial(
        pl.kernel,
        out_type=jax.ShapeDtypeStruct((NJ, KT, 128, KR, GATHER), jnp.float32),
        mesh=mesh,
        compiler_params=pltpu.CompilerParams(use_tc_tiling_on_sc=False),
        scratch_types=[
            pltpu.VMEM((4, IT, GATHER), jnp.int32),
            pltpu.VMEM((4, IT, 1, GATHER, D), jnp.float32),
            pltpu.VMEM((IPW, D), jnp.float32),
            pltpu.SemaphoreType.DMA,
            pltpu.SemaphoreType.DMA,
            pltpu.SemaphoreType.DMA,
            pltpu.SemaphoreType.DMA,
            pltpu.SemaphoreType.DMA,
            pltpu.SemaphoreType.DMA,
            pltpu.SemaphoreType.DMA,
            pltpu.SemaphoreType.DMA,
            pltpu.SemaphoreType.DMA,
            pltpu.SemaphoreType.DMA,
            pltpu.SemaphoreType.DMA,
            pltpu.SemaphoreType.DMA,
        ],
    )
    def k(table_hbm, idx_hbm, out_hbm, idx_v, rows_v, dummy_v,
          i0, i1, i2, i3, g0, g1, g2, g3, s0, s1, s2, s3):
        isem = (i0, i1, i2, i3)
        gsem = (g0, g1, g2, g3)
        ssem = (s0, s1, s2, s3)
        wid = lax.axis_index("s") * NUM_CORES + lax.axis_index("c")
        it0 = pl.multiple_of(wid * IT, IT)

        def fire_idx(j, b):
            pltpu.async_copy(
                idx_hbm.at[j, pl.ds(it0, IT)], idx_v.at[b], isem[b]
            )

        def drain_idx(b):
            pltpu.make_async_copy(
                idx_hbm.at[0, pl.ds(0, IT)], idx_v.at[b], isem[b]
            ).wait()

        def fire_gathers(b):
            for r in range(IT):
                pltpu.async_copy(
                    table_hbm.at[idx_v.at[b, r]],
                    rows_v.at[b, r, 0],
                    gsem[b],
                )

        def drain_gathers(b):
            pltpu.make_async_copy(
                table_hbm.at[pl.ds(0, IPW)], dummy_v, gsem[b]
            ).wait()

        def fire_stores(j, b):
            for kt in range(KT):
                for kr in range(KR):
                    col = kt * KR + kr
                    pltpu.async_copy(
                        rows_v.at[b, :, :, :, col],
                        out_hbm.at[j, kt, pl.ds(it0, IT), pl.ds(kr, 1)],
                        ssem[b],
                    )

        def drain_stores(b):
            pltpu.make_async_copy(
                table_hbm.at[pl.ds(0, IPW)], dummy_v, ssem[b]
            ).wait()

        for j in range(4):
            fire_idx(j, j)
        for j in (0, 1):
            drain_idx(j)
            fire_gathers(j)

        def quad(jq, carry):
            for b in range(4):
                j = 4 * jq + b
                bn = (b + 2) % 4

                drain_gathers(b)
                fire_stores(j, b)

                @pl.when(jq < NJ // 4 - 1)
                def _():
                    fire_idx(j + 4, b)

                def start_next():
                    drain_idx(bn)
                    fire_gathers(bn)

                if b < 2:

                    @pl.when(jq >= 1)
                    def _():
                        drain_stores(bn)

                    start_next()
                else:

                    @pl.when(jq < NJ // 4 - 1)
                    def _():
                        drain_stores(bn)
                        start_next()

            return carry

        lax.fori_loop(0, NJ // 4, quad, 0)
        for b in range(4):
            drain_stores(b)

    return k(table, idx3)


def kernel(indices, weight):
    ni, nj = indices.shape
    idx3 = jnp.transpose(indices).reshape(nj, ni // 128, 128).astype(jnp.int32)
    o5 = _sc_embedding_lookup(weight, idx3)
    out = jnp.transpose(o5, (2, 4, 0, 1, 3)).reshape(ni, nj, D)
    return out


# speedup vs baseline: 94.9302x; 94.9302x over previous
import functools

import jax
import jax.numpy as jnp
from jax import lax
from jax.experimental import pallas as pl
from jax.experimental.pallas import tpu as pltpu
from jax.experimental.pallas import tpu_sc as plsc

NUM_CORES = 2
NUM_SUBCORES = 16
NUM_WORKERS = NUM_CORES * NUM_SUBCORES

CHUNK = 512
OCT = 8


def _sc_gather(table, idx_flat):
    B = idx_flat.shape[0]
    D = table.shape[1]
    rows_per_w = B // NUM_WORKERS
    chunks_per_w = rows_per_w // CHUNK
    nquads = chunks_per_w // 4
    nocts = chunks_per_w // OCT
    assert rows_per_w % (CHUNK * OCT) == 0

    mesh = plsc.VectorSubcoreMesh(core_axis_name="c", subcore_axis_name="s")

    @functools.partial(
        pl.kernel,
        out_type=jax.ShapeDtypeStruct((B, D), jnp.float32),
        mesh=mesh,
        compiler_params=pltpu.CompilerParams(
            use_tc_tiling_on_sc=False, needs_layout_passes=False
        ),
        scratch_types=[
            pltpu.VMEM((2, OCT * CHUNK), jnp.int32),
            pltpu.VMEM((4, CHUNK, D), jnp.float32),
            pltpu.SemaphoreType.DMA,
            pltpu.SemaphoreType.DMA,
            pltpu.SemaphoreType.DMA,
            pltpu.SemaphoreType.DMA,
            pltpu.SemaphoreType.DMA,
            pltpu.SemaphoreType.DMA,
            pltpu.SemaphoreType.DMA,
            pltpu.SemaphoreType.DMA,
            pltpu.SemaphoreType.DMA,
        ],
    )
    def k(table_hbm, idx_hbm, out_hbm, idx_v, rows_v,
          isem, g0, g1, g2, g3, s0, s1, s2, s3):
        gsem = (g0, g1, g2, g3)
        ssem = (s0, s1, s2, s3)
        wid = lax.axis_index("s") * NUM_CORES + lax.axis_index("c")
        row0 = wid * rows_per_w

        def fire_idx_load(o, islot):
            base = pl.multiple_of(row0 + o * OCT * CHUNK, CHUNK)
            pltpu.async_copy(
                idx_hbm.at[pl.ds(base, OCT * CHUNK)], idx_v.at[islot], isem
            )

        def drain_idx_load(islot):
            pltpu.make_async_copy(
                idx_hbm.at[pl.ds(0, OCT * CHUNK)], idx_v.at[islot], isem
            ).wait()

        def fire_gather(islot, orow, b, sem):
            pltpu.async_copy(
                table_hbm.at[idx_v.at[islot, pl.ds(orow * CHUNK, CHUNK)]],
                rows_v.at[b],
                sem,
            )

        def drain_gather(b, sem):
            pltpu.make_async_copy(
                table_hbm.at[pl.ds(0, CHUNK)], rows_v.at[b], sem
            ).wait()

        def fire_store(c, b, sem):
            base = pl.multiple_of(row0 + c * CHUNK, CHUNK)
            pltpu.async_copy(rows_v.at[b], out_hbm.at[pl.ds(base, CHUNK)], sem)

        def drain_store(b, sem):
            pltpu.make_async_copy(
                rows_v.at[b], out_hbm.at[pl.ds(0, CHUNK)], sem
            ).wait()

        fire_idx_load(0, 0)
        drain_idx_load(0)

        def quad(q, carry):
            o = q // 2
            islot = lax.rem(o, 2)
            qh = lax.rem(q, 2)
            even_q = qh == 0
            for b in range(4):
                c = 4 * q + b
                orow = 4 * qh + b

                if b == 0:

                    @pl.when(jnp.logical_and(even_q, q > 0))
                    def _():
                        drain_idx_load(islot)

                @pl.when(q >= 1)
                def _():
                    drain_store(b, ssem[b])

                fire_gather(islot, orow, b, gsem[b])

                if b == 2:

                    @pl.when(jnp.logical_and(even_q, o + 1 < nocts))
                    def _():
                        fire_idx_load(o + 1, 1 - islot)

                bl = (b + 2) % 4
                if b >= 2:
                    drain_gather(bl, gsem[bl])
                    fire_store(c - 2, bl, ssem[bl])
                else:

                    @pl.when(q >= 1)
                    def _():
                        drain_gather(bl, gsem[bl])
                        fire_store(c - 2, bl, ssem[bl])

            return carry

        lax.fori_loop(0, nquads, quad, 0)

        last = chunks_per_w
        for (c, b) in ((last - 2, 2), (last - 1, 3)):
            drain_gather(b, gsem[b])
            fire_store(c, b, ssem[b])
        for b in range(4):
            drain_store(b, ssem[b])

    return k(table, idx_flat)


def kernel(indices, weight):
    B = indices.shape[0] * indices.shape[1]
    idx_flat = indices.reshape(B).astype(jnp.int32)
    out = _sc_gather(weight, idx_flat)
    return out.reshape(indices.shape + (weight.shape[1],))
